# Initial kernel scaffold; baseline (speedup 1.0000x reference)
import jax
import jax.numpy as jnp
from jax.experimental import pallas as pl


def kernel(x, edge_index_pos, edge_index_neg, W_pos_base, b_pos_base, W_neg_base, b_neg_base, W_pos_deep_0, b_pos_deep_0, W_pos_deep_1, b_pos_deep_1, W_neg_deep_0, b_neg_deep_0, W_neg_deep_1, b_neg_deep_1):
    raise NotImplementedError("write your pallas kernel here")


# speedup vs baseline: 10.4256x; 10.4256x over previous
import functools

import jax
import jax.numpy as jnp
from jax import lax
from jax.experimental import pallas as pl
from jax.experimental.pallas import tpu as pltpu
from jax.experimental.pallas import tpu_sc as plsc

N = 50000
D = 128
E = 400000
H = 32

CH = 128
NB = 200
PE = CH * NB
E_PAD = 16 * PE
TOT_NB = 16 * NB
GSTG = 8
K = 4
NST = NB // GSTG
ACC_N = 51200
RPT = ACC_N // 16
NZ = RPT // 128

BLK = 2000
GRID = N // BLK


def _l2n(v):
    nrm = jnp.sqrt(jnp.sum(v * v, axis=1, keepdims=True))
    return v / jnp.maximum(nrm, 1e-12)


def _segsum_body(table, cidx, ridx, out, cbuf, rbuf, rows, acc, gsem, ssem):
    core = lax.axis_index("c")
    sid = lax.axis_index("s")
    zero16 = jnp.zeros((16,), jnp.float32)

    def zb(i, c):
        rows[0, i, pl.ds(0, 16)] = zero16
        rows[0, i, pl.ds(16, 16)] = zero16
        return c
    lax.fori_loop(0, CH, zb, 0)

    row0 = sid * RPT

    def zc(i, c):
        pltpu.sync_copy(rows.at[0], acc.at[pl.ds(row0 + i * CH, CH)])
        return c
    lax.fori_loop(0, NZ, zc, 0)

    nb0 = sid * NB
    plsc.subcore_barrier()

    def stage(st, c):
        j0 = nb0 + st * GSTG
        pltpu.sync_copy(cidx.at[core, pl.ds(j0, GSTG)], cbuf)
        pltpu.sync_copy(ridx.at[core, pl.ds(j0, GSTG)], rbuf)
        for sg in range(GSTG // K):
            gd = [pltpu.async_copy(table.at[cbuf.at[sg * K + b]],
                                   rows.at[b], gsem)
                  for b in range(K)]
            for d in gd:
                d.wait()
            sd = [pltpu.async_copy(rows.at[b],
                                   acc.at[rbuf.at[sg * K + b]], ssem,
                                   add=True)
                  for b in range(K)]
            for d in sd:
                d.wait()
        return c
    lax.fori_loop(0, NST, stage, 0)

    plsc.subcore_barrier()
    pltpu.sync_copy(acc.at[pl.ds(row0, RPT)], out.at[core, pl.ds(row0, RPT)])


def _counts_body(ridx, out, rbuf, ones, zbuf, acc, ssem):
    core = lax.axis_index("c")
    sid = lax.axis_index("s")
    zero16 = jnp.zeros((16,), jnp.float32)
    one16 = jnp.ones((16,), jnp.float32)

    def fill(i, c):
        zbuf[i, pl.ds(0, 16)] = zero16
        ones[i, pl.ds(0, 16)] = one16
        return c
    lax.fori_loop(0, CH, fill, 0)

    row0 = sid * RPT

    def zc(i, c):
        pltpu.sync_copy(zbuf, acc.at[pl.ds(row0 + i * CH, CH)])
        return c
    lax.fori_loop(0, NZ, zc, 0)

    nb0 = sid * NB
    plsc.subcore_barrier()

    def stage(st, c):
        j0 = nb0 + st * GSTG
        pltpu.sync_copy(ridx.at[core, pl.ds(j0, GSTG)], rbuf)
        for sg in range(GSTG // K):
            sd = [pltpu.async_copy(ones, acc.at[rbuf.at[sg * K + b]], ssem,
                                   add=True)
                  for b in range(K)]
            for d in sd:
                d.wait()
        return c
    lax.fori_loop(0, NST, stage, 0)

    plsc.subcore_barrier()
    pltpu.sync_copy(acc.at[pl.ds(row0, RPT)], out.at[core, pl.ds(row0, RPT)])


def _make_sc_kernels():
    mesh = plsc.VectorSubcoreMesh(core_axis_name="c", subcore_axis_name="s")
    params = pltpu.CompilerParams(use_tc_tiling_on_sc=False)
    segsum = pl.kernel(
        _segsum_body,
        out_type=jax.ShapeDtypeStruct((2, ACC_N, H), jnp.float32),
        mesh=mesh,
        compiler_params=params,
        scratch_types=[
            pltpu.VMEM((GSTG, CH), jnp.int32),
            pltpu.VMEM((GSTG, CH), jnp.int32),
            pltpu.VMEM((K, CH, H), jnp.float32),
            pltpu.VMEM_SHARED((ACC_N, H), jnp.float32),
            pltpu.SemaphoreType.DMA,
            pltpu.SemaphoreType.DMA,
        ],
    )
    counts = pl.kernel(
        _counts_body,
        out_type=jax.ShapeDtypeStruct((2, ACC_N, 16), jnp.float32),
        mesh=mesh,
        compiler_params=params,
        scratch_types=[
            pltpu.VMEM((GSTG, CH), jnp.int32),
            pltpu.VMEM((CH, 16), jnp.float32),
            pltpu.VMEM((CH, 16), jnp.float32),
            pltpu.VMEM_SHARED((ACC_N, 16), jnp.float32),
            pltpu.SemaphoreType.DMA,
        ],
    )
    return segsum, counts


def _proj_body(x_ref, w_ref, b_ref, zt_ref, xs_ref):
    y = jnp.dot(x_ref[...], w_ref[...], preferred_element_type=jnp.float32)
    y = y + b_ref[...]
    zt_ref[0] = y[:, 0:H]
    zt_ref[1] = y[:, H:2 * H]
    xs_ref[...] = y[:, 2 * H:]


def _base_body(s_ref, cnt_ref, xs_ref, h_ref):
    dp = jnp.maximum(cnt_ref[0, :, 0:1], 1.0)
    dn = jnp.maximum(cnt_ref[1, :, 0:1], 1.0)
    h_ref[0] = _l2n(s_ref[0] / dp + xs_ref[:, 0:H])
    h_ref[1] = _l2n(s_ref[1] / dn + xs_ref[:, H:])


def _deep_math(sa_ref, sb_ref, cnt_ref, h_ref, wp_ref, bp_ref, wn_ref, bn_ref):
    cp0 = cnt_ref[0, :, 0:1]
    cn0 = cnt_ref[1, :, 0:1]
    dp = jnp.maximum(cp0, 1.0)
    dn = jnp.maximum(cn0, 1.0)
    da = jnp.maximum(cp0 + cn0, 1.0)
    sa0, sa1 = sa_ref[0], sa_ref[1]
    sb0, sb1 = sb_ref[0], sb_ref[1]
    mp_p = sa0 / dp
    mn_p = sa1 / dn
    mp_n = sb0 / dp
    mn_n = sb1 / dn
    ma_p = (sa0 + sa1) / da
    ma_n = (sb0 + sb1) / da
    hp, hn = h_ref[0], h_ref[1]
    catp = jnp.concatenate([mp_p, mn_n, mp_n, mn_p, hp, hn, ma_p], axis=1)
    catn = jnp.concatenate([mn_p, mp_n, mn_n, mp_p, hn, hp, ma_n], axis=1)
    hp2 = _l2n(jnp.dot(catp, wp_ref[...], preferred_element_type=jnp.float32)
               + bp_ref[...])
    hn2 = _l2n(jnp.dot(catn, wn_ref[...], preferred_element_type=jnp.float32)
               + bn_ref[...])
    return hp2, hn2


def _deep_mid_body(sa_ref, sb_ref, cnt_ref, h_ref, wp_ref, bp_ref, wn_ref,
                   bn_ref, o_ref):
    hp2, hn2 = _deep_math(sa_ref, sb_ref, cnt_ref, h_ref, wp_ref, bp_ref,
                          wn_ref, bn_ref)
    o_ref[0] = hp2
    o_ref[1] = hn2


def _deep_final_body(sa_ref, sb_ref, cnt_ref, h_ref, wp_ref, bp_ref, wn_ref,
                     bn_ref, o_ref):
    hp2, hn2 = _deep_math(sa_ref, sb_ref, cnt_ref, h_ref, wp_ref, bp_ref,
                          wn_ref, bn_ref)
    o_ref[...] = jnp.concatenate([hp2, hn2], axis=1)


def _spec_rows(shape):
    if len(shape) == 3:
        return pl.BlockSpec((shape[0], BLK, shape[2]), lambda i: (0, i, 0))
    return pl.BlockSpec((BLK, shape[1]), lambda i: (i, 0))


def _spec_full(shape):
    return pl.BlockSpec(shape, lambda i: (0,) * len(shape))


def _proj_call(x, w, b):
    return pl.pallas_call(
        _proj_body,
        grid=(GRID,),
        in_specs=[_spec_rows(x.shape), _spec_full(w.shape), _spec_full(b.shape)],
        out_specs=[_spec_rows((2, N, H)), _spec_rows((N, 2 * H))],
        out_shape=[jax.ShapeDtypeStruct((2, N, H), jnp.float32),
                   jax.ShapeDtypeStruct((N, 2 * H), jnp.float32)],
    )(x, w, b)


def _base_call(s, cnt, xs):
    return pl.pallas_call(
        _base_body,
        grid=(GRID,),
        in_specs=[_spec_rows(s.shape), _spec_rows(cnt.shape),
                  _spec_rows(xs.shape)],
        out_specs=_spec_rows((2, N, H)),
        out_shape=jax.ShapeDtypeStruct((2, N, H), jnp.float32),
    )(s, cnt, xs)


def _deep_call(body, out_shape, sa, sb, cnt, h, wp, bp, wn, bn):
    return pl.pallas_call(
        body,
        grid=(GRID,),
        in_specs=[_spec_rows(sa.shape), _spec_rows(sb.shape),
                  _spec_rows(cnt.shape), _spec_rows(h.shape),
                  _spec_full(wp.shape), _spec_full(bp.shape),
                  _spec_full(wn.shape), _spec_full(bn.shape)],
        out_specs=_spec_rows(out_shape),
        out_shape=jax.ShapeDtypeStruct(out_shape, jnp.float32),
    )(sa, sb, cnt, h, wp, bp, wn, bn)


def kernel(x, edge_index_pos, edge_index_neg, W_pos_base, b_pos_base,
           W_neg_base, b_neg_base, W_pos_deep_0, b_pos_deep_0, W_pos_deep_1,
           b_pos_deep_1, W_neg_deep_0, b_neg_deep_0, W_neg_deep_1,
           b_neg_deep_1):
    segsum, counts = _make_sc_kernels()

    eip = edge_index_pos.astype(jnp.int32)
    ein = edge_index_neg.astype(jnp.int32)
    rp, cp = eip[0], eip[1]
    rn, cn = ein[0], ein[1]

    pad = E_PAD - E
    cpad = jnp.zeros((pad,), jnp.int32)
    rpad = jnp.full((pad,), N, jnp.int32)
    cpP = jnp.concatenate([cp, cpad])
    cnP = jnp.concatenate([cn, cpad])
    rpP = jnp.concatenate([rp, rpad])
    rnP = jnp.concatenate([rn, rpad])

    ridx = jnp.stack([rpP, rnP]).reshape(2, TOT_NB, CH)
    cidx_base = jnp.stack([cpP, cnP + N]).reshape(2, TOT_NB, CH)
    cidx_a = jnp.stack([cpP, cnP]).reshape(2, TOT_NB, CH)
    cidx_b = jnp.stack([cpP + N, cnP + N]).reshape(2, TOT_NB, CH)

    wcat = jnp.concatenate([W_pos_base[:D], W_neg_base[:D],
                            W_pos_base[D:], W_neg_base[D:]], axis=1)
    bcat = jnp.concatenate([jnp.zeros((2 * H,), jnp.float32),
                            b_pos_base, b_neg_base]).reshape(1, 4 * H)
    zt, xs = _proj_call(x, wcat, bcat)

    cnt = counts(ridx)
    s_base = segsum(zt.reshape(2 * N, H), cidx_base, ridx)
    h = _base_call(s_base, cnt, xs)

    wp = [W_pos_deep_0, W_pos_deep_1]
    bp = [b_pos_deep_0.reshape(1, H), b_pos_deep_1.reshape(1, H)]
    wn = [W_neg_deep_0, W_neg_deep_1]
    bn = [b_neg_deep_0.reshape(1, H), b_neg_deep_1.reshape(1, H)]

    for i in range(2):
        table = h.reshape(2 * N, H)
        sa = segsum(table, cidx_a, ridx)
        sb = segsum(table, cidx_b, ridx)
        if i == 0:
            h = _deep_call(_deep_mid_body, (2, N, H), sa, sb, cnt, h,
                           wp[i], bp[i], wn[i], bn[i])
        else:
            out = _deep_call(_deep_final_body, (N, 2 * H), sa, sb, cnt, h,
                             wp[i], bp[i], wn[i], bn[i])
    return out


# speedup vs baseline: 17.3321x; 1.6624x over previous
import functools

import jax
import jax.numpy as jnp
from jax import lax
from jax.experimental import pallas as pl
from jax.experimental.pallas import tpu as pltpu
from jax.experimental.pallas import tpu_sc as plsc

N = 50000
D = 128
E = 400000
H = 32

CH = 112
NB = 224
PE = CH * NB
E_PAD = 16 * PE
TOT_NB = 16 * NB
K = 4
NGRP = NB // K
NSUP = NGRP // 2
KC = 8
NSTC = NB // KC
ACC_N = 50176
RPT = ACC_N // 16
NZ = RPT // CH

BLK = 2000
GRID = N // BLK


def _l2n(v):
    nrm = jnp.sqrt(jnp.sum(v * v, axis=1, keepdims=True))
    return v / jnp.maximum(nrm, 1e-12)


def _segsum_body(table, cidx, ridx, out, cbk, rbk, rows, acc,
                 gsem0, gsem1, ssem0, ssem1, isem0, isem1):
    core = lax.axis_index("c")
    sid = lax.axis_index("s")
    zero16 = jnp.zeros((16,), jnp.float32)

    def zb(i, c):
        rows[0, 0, i, pl.ds(0, 16)] = zero16
        rows[0, 0, i, pl.ds(16, 16)] = zero16
        return c
    lax.fori_loop(0, CH, zb, 0)

    row0 = sid * RPT

    def zc(i, c):
        pltpu.sync_copy(rows.at[0, 0], acc.at[pl.ds(row0 + i * CH, CH)])
        return c
    lax.fori_loop(0, NZ, zc, 0)

    nb0 = sid * NB
    plsc.subcore_barrier()

    gsems = (gsem0, gsem1)
    ssems = (ssem0, ssem1)
    isems = (isem0, isem1)

    def _gathers(bank, gs):
        return [pltpu.async_copy(table.at[cbk.at[bank, b]],
                                 rows.at[bank, b], gs) for b in range(K)]

    def _drain_gathers(bank):
        for b in range(K):
            pltpu.make_async_copy(table.at[cbk.at[bank, b]],
                                  rows.at[bank, b], gsems[bank]).wait()

    def _scatters(bank):
        return [pltpu.async_copy(rows.at[bank, b],
                                 acc.at[rbk.at[bank, b]], ssems[bank],
                                 add=True)
                for b in range(K)]

    for bank in range(2):
        off = nb0 + bank * K
        pltpu.sync_copy(cidx.at[core, pl.ds(off, K)], cbk.at[bank])
        pltpu.sync_copy(ridx.at[core, pl.ds(off, K)], rbk.at[bank])
        _gathers(bank, gsems[bank])

    def sup(s, c):
        sds = []
        cst = []
        for bank in range(2):
            _drain_gathers(bank)
            sds.append(_scatters(bank))
            nxt = nb0 + (2 * s + 2 + bank) * K
            cst.append(pltpu.async_copy(cidx.at[core, pl.ds(nxt, K)],
                                        cbk.at[bank], isems[bank]))
        rst = []
        for bank in range(2):
            for d in sds[bank]:
                d.wait()
            nxt = nb0 + (2 * s + 2 + bank) * K
            rst.append(pltpu.async_copy(ridx.at[core, pl.ds(nxt, K)],
                                        rbk.at[bank], isems[bank]))
            cst[bank].wait()
            _gathers(bank, gsems[bank])
        for d in rst:
            d.wait()
        return c
    lax.fori_loop(0, NSUP - 1, sup, 0)

    for bank in range(2):
        _drain_gathers(bank)
        for d in _scatters(bank):
            d.wait()

    plsc.subcore_barrier()
    pltpu.sync_copy(acc.at[pl.ds(row0, RPT)], out.at[core, pl.ds(row0, RPT)])


def _counts_body(ridx, out, rbuf, ones, zbuf, acc, ssem):
    core = lax.axis_index("c")
    sid = lax.axis_index("s")
    zero16 = jnp.zeros((16,), jnp.float32)
    one16 = jnp.ones((16,), jnp.float32)

    def fill(i, c):
        zbuf[i, pl.ds(0, 16)] = zero16
        ones[i, pl.ds(0, 16)] = one16
        return c
    lax.fori_loop(0, CH, fill, 0)

    row0 = sid * RPT

    def zc(i, c):
        pltpu.sync_copy(zbuf, acc.at[pl.ds(row0 + i * CH, CH)])
        return c
    lax.fori_loop(0, NZ, zc, 0)

    nb0 = sid * NB
    plsc.subcore_barrier()

    def stage(st, c):
        j0 = nb0 + st * KC
        pltpu.sync_copy(ridx.at[core, pl.ds(j0, KC)], rbuf)
        sd = [pltpu.async_copy(ones, acc.at[rbuf.at[b]], ssem, add=True)
              for b in range(KC)]
        for d in sd:
            d.wait()
        return c
    lax.fori_loop(0, NSTC, stage, 0)

    plsc.subcore_barrier()
    pltpu.sync_copy(acc.at[pl.ds(row0, RPT)], out.at[core, pl.ds(row0, RPT)])


def _make_sc_kernels():
    mesh = plsc.VectorSubcoreMesh(core_axis_name="c", subcore_axis_name="s")
    params = pltpu.CompilerParams(use_tc_tiling_on_sc=False)
    segsum = pl.kernel(
        _segsum_body,
        out_type=jax.ShapeDtypeStruct((2, ACC_N, H), jnp.float32),
        mesh=mesh,
        compiler_params=params,
        scratch_types=[
            pltpu.VMEM((2, K, CH), jnp.int32),
            pltpu.VMEM((2, K, CH), jnp.int32),
            pltpu.VMEM((2, K, CH, H), jnp.float32),
            pltpu.VMEM_SHARED((ACC_N, H), jnp.float32),
            pltpu.SemaphoreType.DMA,
            pltpu.SemaphoreType.DMA,
            pltpu.SemaphoreType.DMA,
            pltpu.SemaphoreType.DMA,
            pltpu.SemaphoreType.DMA,
            pltpu.SemaphoreType.DMA,
        ],
    )
    counts = pl.kernel(
        _counts_body,
        out_type=jax.ShapeDtypeStruct((2, ACC_N, 16), jnp.float32),
        mesh=mesh,
        compiler_params=params,
        scratch_types=[
            pltpu.VMEM((KC, CH), jnp.int32),
            pltpu.VMEM((CH, 16), jnp.float32),
            pltpu.VMEM((CH, 16), jnp.float32),
            pltpu.VMEM_SHARED((ACC_N, 16), jnp.float32),
            pltpu.SemaphoreType.DMA,
        ],
    )
    return segsum, counts


def _proj_body(x_ref, w_ref, b_ref, zt_ref, xs_ref):
    y = jnp.dot(x_ref[...], w_ref[...], preferred_element_type=jnp.float32)
    y = y + b_ref[...]
    zt_ref[0] = y[:, 0:H]
    zt_ref[1] = y[:, H:2 * H]
    xs_ref[...] = y[:, 2 * H:]


def _base_body(s_ref, cnt_ref, xs_ref, h_ref):
    dp = jnp.maximum(cnt_ref[0, :, 0:1], 1.0)
    dn = jnp.maximum(cnt_ref[1, :, 0:1], 1.0)
    h_ref[0] = _l2n(s_ref[0] / dp + xs_ref[:, 0:H])
    h_ref[1] = _l2n(s_ref[1] / dn + xs_ref[:, H:])


def _deep_math(sa_ref, sb_ref, cnt_ref, h_ref, wp_ref, bp_ref, wn_ref, bn_ref):
    cp0 = cnt_ref[0, :, 0:1]
    cn0 = cnt_ref[1, :, 0:1]
    dp = jnp.maximum(cp0, 1.0)
    dn = jnp.maximum(cn0, 1.0)
    da = jnp.maximum(cp0 + cn0, 1.0)
    sa0, sa1 = sa_ref[0], sa_ref[1]
    sb0, sb1 = sb_ref[0], sb_ref[1]
    mp_p = sa0 / dp
    mn_p = sa1 / dn
    mp_n = sb0 / dp
    mn_n = sb1 / dn
    ma_p = (sa0 + sa1) / da
    ma_n = (sb0 + sb1) / da
    hp, hn = h_ref[0], h_ref[1]
    catp = jnp.concatenate([mp_p, mn_n, mp_n, mn_p, hp, hn, ma_p], axis=1)
    catn = jnp.concatenate([mn_p, mp_n, mn_n, mp_p, hn, hp, ma_n], axis=1)
    hp2 = _l2n(jnp.dot(catp, wp_ref[...], preferred_element_type=jnp.float32)
               + bp_ref[...])
    hn2 = _l2n(jnp.dot(catn, wn_ref[...], preferred_element_type=jnp.float32)
               + bn_ref[...])
    return hp2, hn2


def _deep_mid_body(sa_ref, sb_ref, cnt_ref, h_ref, wp_ref, bp_ref, wn_ref,
                   bn_ref, o_ref):
    hp2, hn2 = _deep_math(sa_ref, sb_ref, cnt_ref, h_ref, wp_ref, bp_ref,
                          wn_ref, bn_ref)
    o_ref[0] = hp2
    o_ref[1] = hn2


def _deep_final_body(sa_ref, sb_ref, cnt_ref, h_ref, wp_ref, bp_ref, wn_ref,
                     bn_ref, o_ref):
    hp2, hn2 = _deep_math(sa_ref, sb_ref, cnt_ref, h_ref, wp_ref, bp_ref,
                          wn_ref, bn_ref)
    o_ref[...] = jnp.concatenate([hp2, hn2], axis=1)


def _spec_rows(shape):
    if len(shape) == 3:
        return pl.BlockSpec((shape[0], BLK, shape[2]), lambda i: (0, i, 0))
    return pl.BlockSpec((BLK, shape[1]), lambda i: (i, 0))


def _spec_full(shape):
    return pl.BlockSpec(shape, lambda i: (0,) * len(shape))


def _proj_call(x, w, b):
    return pl.pallas_call(
        _proj_body,
        grid=(GRID,),
        in_specs=[_spec_rows(x.shape), _spec_full(w.shape), _spec_full(b.shape)],
        out_specs=[_spec_rows((2, N, H)), _spec_rows((N, 2 * H))],
        out_shape=[jax.ShapeDtypeStruct((2, N, H), jnp.float32),
                   jax.ShapeDtypeStruct((N, 2 * H), jnp.float32)],
    )(x, w, b)


def _base_call(s, cnt, xs):
    return pl.pallas_call(
        _base_body,
        grid=(GRID,),
        in_specs=[_spec_rows(s.shape), _spec_rows(cnt.shape),
                  _spec_rows(xs.shape)],
        out_specs=_spec_rows((2, N, H)),
        out_shape=jax.ShapeDtypeStruct((2, N, H), jnp.float32),
    )(s, cnt, xs)


def _deep_call(body, out_shape, sa, sb, cnt, h, wp, bp, wn, bn):
    return pl.pallas_call(
        body,
        grid=(GRID,),
        in_specs=[_spec_rows(sa.shape), _spec_rows(sb.shape),
                  _spec_rows(cnt.shape), _spec_rows(h.shape),
                  _spec_full(wp.shape), _spec_full(bp.shape),
                  _spec_full(wn.shape), _spec_full(bn.shape)],
        out_specs=_spec_rows(out_shape),
        out_shape=jax.ShapeDtypeStruct(out_shape, jnp.float32),
    )(sa, sb, cnt, h, wp, bp, wn, bn)


def kernel(x, edge_index_pos, edge_index_neg, W_pos_base, b_pos_base,
           W_neg_base, b_neg_base, W_pos_deep_0, b_pos_deep_0, W_pos_deep_1,
           b_pos_deep_1, W_neg_deep_0, b_neg_deep_0, W_neg_deep_1,
           b_neg_deep_1):
    segsum, counts = _make_sc_kernels()

    eip = edge_index_pos.astype(jnp.int32)
    ein = edge_index_neg.astype(jnp.int32)
    rp, cp = eip[0], eip[1]
    rn, cn = ein[0], ein[1]

    pad = E_PAD - E
    cpad = jnp.zeros((pad,), jnp.int32)
    rpad = jnp.full((pad,), N, jnp.int32)
    cpP = jnp.concatenate([cp, cpad])
    cnP = jnp.concatenate([cn, cpad])
    rpP = jnp.concatenate([rp, rpad])
    rnP = jnp.concatenate([rn, rpad])

    ridx = jnp.stack([rpP, rnP]).reshape(2, TOT_NB, CH)
    cidx_base = jnp.stack([cpP, cnP + N]).reshape(2, TOT_NB, CH)
    cidx_a = jnp.stack([cpP, cnP]).reshape(2, TOT_NB, CH)
    cidx_b = jnp.stack([cpP + N, cnP + N]).reshape(2, TOT_NB, CH)

    wcat = jnp.concatenate([W_pos_base[:D], W_neg_base[:D],
                            W_pos_base[D:], W_neg_base[D:]], axis=1)
    bcat = jnp.concatenate([jnp.zeros((2 * H,), jnp.float32),
                            b_pos_base, b_neg_base]).reshape(1, 4 * H)
    zt, xs = _proj_call(x, wcat, bcat)

    cnt = counts(ridx)
    s_base = segsum(zt.reshape(2 * N, H), cidx_base, ridx)
    h = _base_call(s_base, cnt, xs)

    wp = [W_pos_deep_0, W_pos_deep_1]
    bp = [b_pos_deep_0.reshape(1, H), b_pos_deep_1.reshape(1, H)]
    wn = [W_neg_deep_0, W_neg_deep_1]
    bn = [b_neg_deep_0.reshape(1, H), b_neg_deep_1.reshape(1, H)]

    for i in range(2):
        table = h.reshape(2 * N, H)
        sa = segsum(table, cidx_a, ridx)
        sb = segsum(table, cidx_b, ridx)
        if i == 0:
            h = _deep_call(_deep_mid_body, (2, N, H), sa, sb, cnt, h,
                           wp[i], bp[i], wn[i], bn[i])
        else:
            out = _deep_call(_deep_final_body, (N, 2 * H), sa, sb, cnt, h,
                             wp[i], bp[i], wn[i], bn[i])
    return out
